# baseline (speedup 1.0000x reference)
import functools

import jax
import jax.numpy as jnp
from jax import lax
from jax.experimental import pallas as pl
from jax.experimental.pallas import tpu as pltpu
from jax.experimental.pallas import tpu_sc as plsc

NUM_EXPERTS = 64
TOP_K = 8
HIDDEN = 4096
TOKENS = 16384
BLOCK_T = 1024
N_CHUNKS = 2
CHUNK_T = TOKENS // N_CHUNKS

_INFO = plsc.get_sparse_core_info()
NC, NS, L = _INFO.num_cores, _INFO.num_subcores, _INFO.num_lanes
NW = NC * NS


def _gating_block(x_ref, w_ref, logits_ref, probs_t_ref):
    x = x_ref[...]
    w = w_ref[...]
    logits_t = jax.lax.dot_general(
        w, x, (((1,), (1,)), ((), ())), preferred_element_type=jnp.float32
    )
    logits_ref[...] = logits_t.T
    m = jnp.max(logits_t, axis=0, keepdims=True)
    e = jnp.exp(logits_t - m)
    probs_t_ref[...] = e / jnp.sum(e, axis=0, keepdims=True)


def _gating(x_chunk, weight):
    grid = (CHUNK_T // BLOCK_T,)
    return pl.pallas_call(
        _gating_block,
        grid=grid,
        in_specs=[
            pl.BlockSpec((BLOCK_T, HIDDEN), lambda i: (i, 0)),
            pl.BlockSpec((NUM_EXPERTS, HIDDEN), lambda i: (0, 0)),
        ],
        out_specs=[
            pl.BlockSpec((BLOCK_T, NUM_EXPERTS), lambda i: (i, 0)),
            pl.BlockSpec((NUM_EXPERTS, BLOCK_T), lambda i: (0, i)),
        ],
        out_shape=[
            jax.ShapeDtypeStruct((CHUNK_T, NUM_EXPERTS), jnp.float32),
            jax.ShapeDtypeStruct((NUM_EXPERTS, CHUNK_T), jnp.float32),
        ],
    )(x_chunk, weight)


def _make_topk_sc(tokens):
    tpw = tokens // NW
    groups = tpw // L

    @functools.partial(
        pl.kernel,
        out_type=[
            jax.ShapeDtypeStruct((TOP_K, tokens), jnp.float32),
            jax.ShapeDtypeStruct((TOP_K, tokens), jnp.int32),
        ],
        mesh=plsc.VectorSubcoreMesh(core_axis_name="c", subcore_axis_name="s"),
        scratch_types=[
            pltpu.VMEM((NUM_EXPERTS, tpw), jnp.float32),
            pltpu.VMEM((TOP_K, tpw), jnp.float32),
            pltpu.VMEM((TOP_K, tpw), jnp.int32),
        ],
    )
    def _topk_sc(probs_t_hbm, scores_hbm, idx_hbm, probs_v, out_s, out_i):
        wid = lax.axis_index("s") * NC + lax.axis_index("c")
        base = wid * tpw
        pltpu.sync_copy(probs_t_hbm.at[:, pl.ds(base, tpw)], probs_v)

        def per_group(g, carry):
            vals = [jnp.full((L,), -1.0, jnp.float32) for _ in range(TOP_K)]
            idxs = [jnp.full((L,), 0, jnp.int32) for _ in range(TOP_K)]
            col = g * L
            for e in range(NUM_EXPERTS):
                v = probs_v[e, pl.ds(col, L)]
                i = jnp.full((L,), e, jnp.int32)
                for j in range(TOP_K):
                    c = v > vals[j]
                    vals[j], v = (
                        jnp.where(c, v, vals[j]),
                        jnp.where(c, vals[j], v),
                    )
                    idxs[j], i = (
                        jnp.where(c, i, idxs[j]),
                        jnp.where(c, idxs[j], i),
                    )
            for j in range(TOP_K):
                out_s[j, pl.ds(col, L)] = vals[j]
                out_i[j, pl.ds(col, L)] = idxs[j]
            return carry

        lax.fori_loop(0, groups, per_group, 0)
        pltpu.sync_copy(out_s, scores_hbm.at[:, pl.ds(base, tpw)])
        pltpu.sync_copy(out_i, idx_hbm.at[:, pl.ds(base, tpw)])

    return _topk_sc


_topk_chunk = _make_topk_sc(CHUNK_T)


@jax.jit
def kernel(input, weight):
    scores_t, idx_t, logits = [], [], []
    for k in range(N_CHUNKS):
        lg, pt = _gating(
            lax.slice_in_dim(input, k * CHUNK_T, (k + 1) * CHUNK_T, axis=0),
            weight,
        )
        s, i = _topk_chunk(pt)
        logits.append(lg)
        scores_t.append(s)
        idx_t.append(i)
    top_scores = jnp.concatenate(scores_t, axis=1).T
    top_indices = jnp.concatenate(idx_t, axis=1).T
    return top_scores, top_indices, jnp.concatenate(logits, axis=0)


# speedup vs baseline: 2.3136x; 2.3136x over previous
import functools

import jax
import jax.numpy as jnp
from jax import lax
from jax.experimental import pallas as pl
from jax.experimental.pallas import tpu as pltpu
from jax.experimental.pallas import tpu_sc as plsc

NUM_EXPERTS = 64
TOP_K = 8
HIDDEN = 4096
TOKENS = 16384
BLOCK_T = 1024
N_CHUNKS = 2
CHUNK_T = TOKENS // N_CHUNKS

_INFO = plsc.get_sparse_core_info()
NC, NS, L = _INFO.num_cores, _INFO.num_subcores, _INFO.num_lanes
NW = NC * NS


def _gating_block(x_ref, w_ref, logits_ref, probs_t_ref):
    x = x_ref[...]
    w = w_ref[...]
    logits_t = jax.lax.dot_general(
        w, x, (((1,), (1,)), ((), ())), preferred_element_type=jnp.float32
    )
    logits_ref[...] = logits_t.T
    m = jnp.max(logits_t, axis=0, keepdims=True)
    e = jnp.exp(logits_t - m)
    probs_t_ref[...] = e / jnp.sum(e, axis=0, keepdims=True)


def _gating(x_full, weight, chunk):
    grid = (CHUNK_T // BLOCK_T,)
    off = chunk * (CHUNK_T // BLOCK_T)
    return pl.pallas_call(
        _gating_block,
        grid=grid,
        in_specs=[
            pl.BlockSpec((BLOCK_T, HIDDEN), lambda i: (off + i, 0)),
            pl.BlockSpec((NUM_EXPERTS, HIDDEN), lambda i: (0, 0)),
        ],
        out_specs=[
            pl.BlockSpec((BLOCK_T, NUM_EXPERTS), lambda i: (i, 0)),
            pl.BlockSpec((NUM_EXPERTS, BLOCK_T), lambda i: (0, i)),
        ],
        out_shape=[
            jax.ShapeDtypeStruct((CHUNK_T, NUM_EXPERTS), jnp.float32),
            jax.ShapeDtypeStruct((NUM_EXPERTS, CHUNK_T), jnp.float32),
        ],
    )(x_full, weight)


def _make_topk_sc(tokens):
    tpw = tokens // NW
    groups = tpw // L

    @functools.partial(
        pl.kernel,
        out_type=[
            jax.ShapeDtypeStruct((TOP_K, tokens), jnp.float32),
            jax.ShapeDtypeStruct((TOP_K, tokens), jnp.int32),
        ],
        mesh=plsc.VectorSubcoreMesh(core_axis_name="c", subcore_axis_name="s"),
        scratch_types=[
            pltpu.VMEM((NUM_EXPERTS, tpw), jnp.float32),
            pltpu.VMEM((TOP_K, tpw), jnp.float32),
            pltpu.VMEM((TOP_K, tpw), jnp.int32),
        ],
    )
    def _topk_sc(probs_t_hbm, scores_hbm, idx_hbm, probs_v, out_s, out_i):
        wid = lax.axis_index("s") * NC + lax.axis_index("c")
        base = wid * tpw
        pltpu.sync_copy(probs_t_hbm.at[:, pl.ds(base, tpw)], probs_v)

        def per_group(g, carry):
            vals = [jnp.full((L,), -1.0, jnp.float32) for _ in range(TOP_K)]
            idxs = [jnp.full((L,), 0, jnp.int32) for _ in range(TOP_K)]
            col = g * L
            for e in range(NUM_EXPERTS):
                v = probs_v[e, pl.ds(col, L)]
                i = jnp.full((L,), e, jnp.int32)
                for j in range(TOP_K):
                    c = v > vals[j]
                    vals[j], v = (
                        jnp.where(c, v, vals[j]),
                        jnp.where(c, vals[j], v),
                    )
                    idxs[j], i = (
                        jnp.where(c, i, idxs[j]),
                        jnp.where(c, idxs[j], i),
                    )
            for j in range(TOP_K):
                out_s[j, pl.ds(col, L)] = vals[j]
                out_i[j, pl.ds(col, L)] = idxs[j]
            return carry

        lax.fori_loop(0, groups, per_group, 0)
        pltpu.sync_copy(out_s, scores_hbm.at[:, pl.ds(base, tpw)])
        pltpu.sync_copy(out_i, idx_hbm.at[:, pl.ds(base, tpw)])

    return _topk_sc


_topk_chunk = _make_topk_sc(CHUNK_T)


@jax.jit
def kernel(input, weight):
    scores_t, idx_t, logits = [], [], []
    for k in range(N_CHUNKS):
        lg, pt = _gating(input, weight, k)
        s, i = _topk_chunk(pt)
        logits.append(lg)
        scores_t.append(s)
        idx_t.append(i)
    top_scores = jnp.concatenate(scores_t, axis=1).T
    top_indices = jnp.concatenate(idx_t, axis=1).T
    return top_scores, top_indices, jnp.concatenate(logits, axis=0)


# speedup vs baseline: 2.6504x; 1.1456x over previous
import jax
import jax.numpy as jnp
from jax.experimental import pallas as pl
from jax.experimental.pallas import tpu as pltpu

NUM_EXPERTS = 64
TOP_K = 8
HIDDEN = 4096
TOKENS = 16384
BLOCK_T = 1024
N_BLOCKS = TOKENS // BLOCK_T


def _router_block(x_ref, w_ref, scores_ref, idx_ref, logits_ref, lt_buf):
    i = pl.program_id(0)

    @pl.when(i < N_BLOCKS)
    def _matmul():
        logits_t = jax.lax.dot_general(
            w_ref[...],
            x_ref[...],
            (((1,), (1,)), ((), ())),
            preferred_element_type=jnp.float32,
        )
        logits_ref[...] = logits_t.T
        lt_buf[i % 2] = logits_t

    @pl.when(i > 0)
    def _select():
        logits_t = lt_buf[(i + 1) % 2]
        m = jnp.max(logits_t, axis=0, keepdims=True)
        s = jnp.sum(jnp.exp(logits_t - m), axis=0, keepdims=True)
        eidx = jax.lax.broadcasted_iota(jnp.int32, logits_t.shape, 0).astype(
            jnp.float32
        )
        work = logits_t
        cms = []
        cis = []
        for _ in range(TOP_K):
            cm = jnp.max(work, axis=0, keepdims=True)
            ci = jnp.min(
                jnp.where(work == cm, eidx, float(NUM_EXPERTS)),
                axis=0,
                keepdims=True,
            )
            cms.append(cm)
            cis.append(ci)
            work = jnp.where(eidx == ci, -jnp.inf, work)
        cms8 = jnp.concatenate(cms, axis=0)
        cis8 = jnp.concatenate(cis, axis=0)
        scores_ref[...] = (jnp.exp(cms8 - m) / s).T
        idx_ref[...] = cis8.T.astype(jnp.int32)


@jax.jit
def kernel(input, weight):
    grid = (N_BLOCKS + 1,)
    return pl.pallas_call(
        _router_block,
        grid=grid,
        in_specs=[
            pl.BlockSpec((BLOCK_T, HIDDEN), lambda i: (jnp.minimum(i, N_BLOCKS - 1), 0)),
            pl.BlockSpec((NUM_EXPERTS, HIDDEN), lambda i: (0, 0)),
        ],
        out_specs=[
            pl.BlockSpec((BLOCK_T, TOP_K), lambda i: (jnp.maximum(i - 1, 0), 0)),
            pl.BlockSpec((BLOCK_T, TOP_K), lambda i: (jnp.maximum(i - 1, 0), 0)),
            pl.BlockSpec((BLOCK_T, NUM_EXPERTS), lambda i: (jnp.minimum(i, N_BLOCKS - 1), 0)),
        ],
        out_shape=[
            jax.ShapeDtypeStruct((TOKENS, TOP_K), jnp.float32),
            jax.ShapeDtypeStruct((TOKENS, TOP_K), jnp.int32),
            jax.ShapeDtypeStruct((TOKENS, NUM_EXPERTS), jnp.float32),
        ],
        scratch_shapes=[
            pltpu.VMEM((2, NUM_EXPERTS, BLOCK_T), jnp.float32),
        ],
    )(input, weight)


# speedup vs baseline: 2.6849x; 1.0130x over previous
import jax
import jax.numpy as jnp
from jax.experimental import pallas as pl

NUM_EXPERTS = 64
TOP_K = 8
HIDDEN = 4096
TOKENS = 16384
BLOCK_T = 1024


def _router_block(x_ref, w_ref, scores_ref, idx_ref, logits_ref):
    x = x_ref[...]
    w = w_ref[...]
    logits_t = jax.lax.dot_general(
        w, x, (((1,), (1,)), ((), ())), preferred_element_type=jnp.float32
    )
    logits_ref[...] = logits_t.T

    m = jnp.max(logits_t, axis=0, keepdims=True)
    s = jnp.sum(jnp.exp(logits_t - m), axis=0, keepdims=True)

    eidx = jax.lax.broadcasted_iota(jnp.int32, logits_t.shape, 0).astype(
        jnp.float32
    )
    work = logits_t
    cms = []
    cis = []
    for _ in range(TOP_K):
        cm = jnp.max(work, axis=0, keepdims=True)
        ci = jnp.min(
            jnp.where(work == cm, eidx, float(NUM_EXPERTS)), axis=0, keepdims=True
        )
        cms.append(cm)
        cis.append(ci)
        work = jnp.where(eidx == ci, -jnp.inf, work)
    cms8 = jnp.concatenate(cms, axis=0)
    cis8 = jnp.concatenate(cis, axis=0)
    scores_ref[...] = (jnp.exp(cms8 - m) / s).T
    idx_ref[...] = cis8.T.astype(jnp.int32)


@jax.jit
def kernel(input, weight):
    grid = (TOKENS // BLOCK_T,)
    return pl.pallas_call(
        _router_block,
        grid=grid,
        in_specs=[
            pl.BlockSpec((BLOCK_T, HIDDEN), lambda i: (i, 0)),
            pl.BlockSpec((NUM_EXPERTS, HIDDEN), lambda i: (0, 0)),
        ],
        out_specs=[
            pl.BlockSpec((BLOCK_T, TOP_K), lambda i: (i, 0)),
            pl.BlockSpec((BLOCK_T, TOP_K), lambda i: (i, 0)),
            pl.BlockSpec((BLOCK_T, NUM_EXPERTS), lambda i: (i, 0)),
        ],
        out_shape=[
            jax.ShapeDtypeStruct((TOKENS, TOP_K), jnp.float32),
            jax.ShapeDtypeStruct((TOKENS, TOP_K), jnp.int32),
            jax.ShapeDtypeStruct((TOKENS, NUM_EXPERTS), jnp.float32),
        ],
    )(input, weight)


# speedup vs baseline: 2.7230x; 1.0142x over previous
import jax
import jax.numpy as jnp
from jax.experimental import pallas as pl

NUM_EXPERTS = 64
TOP_K = 8
HIDDEN = 4096
TOKENS = 16384
BLOCK_T = 1024


def _router_block(x_ref, w_ref, scores_ref, idx_ref, logits_ref):
    x = x_ref[...]
    w = w_ref[...]
    logits_t = jax.lax.dot_general(
        w, x, (((1,), (1,)), ((), ())), preferred_element_type=jnp.float32
    )
    logits_ref[...] = logits_t.T

    m = jnp.max(logits_t, axis=0, keepdims=True)
    e = jnp.exp(logits_t - m)
    probs_t = e / jnp.sum(e, axis=0, keepdims=True)

    eidx = jax.lax.broadcasted_iota(jnp.int32, logits_t.shape, 0).astype(
        jnp.float32
    )
    work = probs_t
    cms = []
    cis = []
    for _ in range(TOP_K):
        cm = jnp.max(work, axis=0, keepdims=True)
        ci = jnp.min(
            jnp.where(work == cm, eidx, float(NUM_EXPERTS)), axis=0, keepdims=True
        )
        cms.append(cm)
        cis.append(ci)
        work = jnp.where(eidx == ci, -1.0, work)
    scores_ref[...] = jnp.concatenate(cms, axis=0).T
    idx_ref[...] = jnp.concatenate(cis, axis=0).T.astype(jnp.int32)


@jax.jit
def kernel(input, weight):
    grid = (TOKENS // BLOCK_T,)
    return pl.pallas_call(
        _router_block,
        grid=grid,
        in_specs=[
            pl.BlockSpec((BLOCK_T, HIDDEN), lambda i: (i, 0)),
            pl.BlockSpec((NUM_EXPERTS, HIDDEN), lambda i: (0, 0)),
        ],
        out_specs=[
            pl.BlockSpec((BLOCK_T, TOP_K), lambda i: (i, 0)),
            pl.BlockSpec((BLOCK_T, TOP_K), lambda i: (i, 0)),
            pl.BlockSpec((BLOCK_T, NUM_EXPERTS), lambda i: (i, 0)),
        ],
        out_shape=[
            jax.ShapeDtypeStruct((TOKENS, TOP_K), jnp.float32),
            jax.ShapeDtypeStruct((TOKENS, TOP_K), jnp.int32),
            jax.ShapeDtypeStruct((TOKENS, NUM_EXPERTS), jnp.float32),
        ],
    )(input, weight)
